# Initial kernel scaffold; baseline (speedup 1.0000x reference)
import jax
import jax.numpy as jnp
from jax.experimental import pallas as pl


def kernel(images, layer_rect, edges, bbox, labels, node_indices, W_pos, b_pos, W_img, b_img, W_edge, b_edge, W_g1, b_g1, W_g2, b_g2, W_c1, b_c1, W_c2, b_c2, W_l1, b_l1, W_l2, b_l2):
    raise NotImplementedError("write your pallas kernel here")


# speedup vs baseline: 3.8010x; 3.8010x over previous
import functools

import jax
import jax.numpy as jnp
from jax import lax
from jax.experimental import pallas as pl
from jax.experimental.pallas import tpu as pltpu
from jax.experimental.pallas import tpu_sc as plsc

N = 10000
E = 320000
D = 128
NUM_CLASSES = 25

NC = 2
NS = 16
NW = NC * NS
CSZ = 128
CH = 79
EW = CH * CSZ
EP = NW * EW
AGG_ROWS = 10112
ROWS_PER_TILE = AGG_ROWS // NS

@functools.cache
def _sc_mesh():
    return plsc.VectorSubcoreMesh(core_axis_name="c", subcore_axis_name="s",
                                  num_cores=NC, num_subcores=NS)


def _edge_dist_body(rect_hbm, xi_hbm, xj_hbm, dT_hbm,
                    rect_v, xi_v, xj_v, dbuf):
    wid = lax.axis_index("c") * NS + lax.axis_index("s")
    pltpu.sync_copy(rect_hbm, rect_v)
    pltpu.sync_copy(xi_hbm.at[wid], xi_v)
    pltpu.sync_copy(xj_hbm.at[wid], xj_v)

    def row_body(r, carry):
        for j in range(8):
            s = slice(j * 16, (j + 1) * 16)
            vi = xi_v[r, s] * 4
            vj = xj_v[r, s] * 4
            for c in range(4):
                a = plsc.load_gather(rect_v, [vi + c])
                b = plsc.load_gather(rect_v, [vj + c])
                dbuf[c, pl.ds(r * CSZ + j * 16, 16)] = jnp.abs(a - b)
        return carry

    lax.fori_loop(0, CH, row_body, 0)
    pltpu.sync_copy(dbuf, dT_hbm.at[:, pl.ds(wid * EW, EW)])


def _edge_dist(rect_flat, xi_w, xj_w):
    return pl.kernel(
        _edge_dist_body,
        out_type=jax.ShapeDtypeStruct((4, EP), jnp.float32),
        mesh=_sc_mesh(),
        scratch_types=[
            pltpu.VMEM((4 * N,), jnp.float32),
            pltpu.VMEM((CH, CSZ), jnp.int32),
            pltpu.VMEM((CH, CSZ), jnp.int32),
            pltpu.VMEM((4, EW), jnp.float32),
        ],
        compiler_params=pltpu.CompilerParams(needs_layout_passes=False),
    )(rect_flat, xi_w, xj_w)


def _msg_pass_body(x_hbm, e_hbm, xi_hbm, xj_hbm, parts_hbm,
                   xi_r, xj_v, xg, ev, aggr_sh, sem):
    core = lax.axis_index("c")
    sub = lax.axis_index("s")
    wid = core * NS + sub
    pltpu.sync_copy(xj_hbm.at[wid], xj_v)

    zeros16 = jnp.zeros((16,), jnp.float32)

    def zrow(r, carry):
        for j in range(8):
            ev[r, j * 16:(j + 1) * 16] = zeros16
        return carry

    lax.fori_loop(0, CSZ, zrow, 0)
    base = sub * ROWS_PER_TILE
    for b in range(ROWS_PER_TILE // CSZ):
        pltpu.sync_copy(ev, aggr_sh.at[pl.ds(base + b * CSZ, CSZ)])
    rem = ROWS_PER_TILE % CSZ
    if rem:
        pltpu.sync_copy(ev.at[pl.ds(0, rem)],
                        aggr_sh.at[pl.ds(base + (ROWS_PER_TILE // CSZ) * CSZ, rem)])
    plsc.subcore_barrier()

    def chunk_body(c, carry):
        pltpu.sync_copy(xi_hbm.at[wid, c], xi_r)
        pltpu.async_copy(x_hbm.at[xi_r], xg, sem).wait()
        pltpu.sync_copy(e_hbm.at[pl.ds(wid * EW + c * CSZ, CSZ)], ev)

        def row_body(r, rcarry):
            for j in range(8):
                s = slice(j * 16, (j + 1) * 16)
                ev[r, s] = jnp.maximum(ev[r, s] + xg[r, s], 0.0)
            return rcarry

        lax.fori_loop(0, CSZ, row_body, 0)
        pltpu.sync_copy(ev, aggr_sh.at[xj_v.at[c]], add=True)
        return carry

    lax.fori_loop(0, CH, chunk_body, 0)
    plsc.subcore_barrier()
    pltpu.sync_copy(aggr_sh.at[pl.ds(base, ROWS_PER_TILE)],
                    parts_hbm.at[core, pl.ds(base, ROWS_PER_TILE)])


def _msg_pass(x, e, xi_w, xj_w):
    return pl.kernel(
        _msg_pass_body,
        out_type=jax.ShapeDtypeStruct((NC, AGG_ROWS, D), jnp.float32),
        mesh=_sc_mesh(),
        scratch_types=[
            pltpu.VMEM((CSZ,), jnp.int32),
            pltpu.VMEM((CH, CSZ), jnp.int32),
            pltpu.VMEM((CSZ, D), jnp.float32),
            pltpu.VMEM((CSZ, D), jnp.float32),
            pltpu.VMEM_SHARED((AGG_ROWS, D), jnp.float32),
            pltpu.SemaphoreType.DMA,
        ],
        compiler_params=pltpu.CompilerParams(needs_layout_passes=False),
    )(x, e, xi_w, xj_w)


def _embed_body(rect_ref, img_ref, Wp_ref, Wi_ref, b_ref, x_ref):
    r = rect_ref[...]
    feats = [r]
    for i in range(6):
        f = 2.0 ** i
        feats.append(jnp.sin(r * f))
        feats.append(jnp.cos(r * f))
    ft = jnp.concatenate(feats, axis=-1)
    pos = jnp.dot(ft, Wp_ref[...], preferred_element_type=jnp.float32)
    img = jnp.dot(img_ref[...], Wi_ref[...], preferred_element_type=jnp.float32)
    x_ref[...] = pos + img + b_ref[...]


def _embed(rect, images, W_pos, W_img, b):
    BR = 400
    return pl.pallas_call(
        _embed_body,
        grid=(N // BR,),
        in_specs=[
            pl.BlockSpec((BR, 4), lambda i: (i, 0)),
            pl.BlockSpec((BR, 768), lambda i: (i, 0)),
            pl.BlockSpec((52, D), lambda i: (0, 0)),
            pl.BlockSpec((768, D), lambda i: (0, 0)),
            pl.BlockSpec((1, D), lambda i: (0, 0)),
        ],
        out_specs=pl.BlockSpec((BR, D), lambda i: (i, 0)),
        out_shape=jax.ShapeDtypeStruct((N, D), jnp.float32),
    )(rect, images, W_pos, W_img, b)


def _edge_mlp_body(dT_ref, W_ref, b_ref, e_ref):
    d = dT_ref[...]
    feats = [d]
    for i in range(3):
        f = 2.0 ** i
        feats.append(jnp.sin(d * f))
        feats.append(jnp.cos(d * f))
    ft = jnp.concatenate(feats, axis=0)
    e_ref[...] = lax.dot_general(ft, W_ref[...], (((0,), (0,)), ((), ())),
                                 preferred_element_type=jnp.float32) + b_ref[...]


def _edge_mlp(dT, W_edge, b_edge):
    BC = 1024
    return pl.pallas_call(
        _edge_mlp_body,
        grid=(EP // BC,),
        in_specs=[
            pl.BlockSpec((4, BC), lambda i: (0, i)),
            pl.BlockSpec((28, D), lambda i: (0, 0)),
            pl.BlockSpec((1, D), lambda i: (0, 0)),
        ],
        out_specs=pl.BlockSpec((BC, D), lambda i: (i, 0)),
        out_shape=jax.ShapeDtypeStruct((EP, D), jnp.float32),
    )(dT, W_edge, b_edge)


def _tail_body(x_ref, parts_ref, bbox_ref, lab_ref,
               Wg1, bg1, Wg2, bg2, Wc1, bc1, Wc2, bc2, Wl1, bl1, Wl2, bl2,
               logits_ref, pred_ref, tot_ref, cls_ref, reg_ref):
    x = x_ref[...]
    aggr = parts_ref[0, :N, :] + parts_ref[1, :N, :]
    h = jnp.maximum(jnp.dot(x + aggr, Wg1[...], preferred_element_type=jnp.float32)
                    + bg1[...], 0.0)
    gnn = jnp.dot(h, Wg2[...], preferred_element_type=jnp.float32) + bg2[...]
    ch = jnp.maximum(jnp.dot(gnn, Wc1[...], preferred_element_type=jnp.float32)
                     + bc1[...], 0.0)
    logits = jnp.dot(ch, Wc2[...], preferred_element_type=jnp.float32) + bc2[...]
    lh = jnp.maximum(jnp.dot(gnn, Wl1[...], preferred_element_type=jnp.float32)
                     + bl1[...], 0.0)
    pred = jnp.dot(lh, Wl2[...], preferred_element_type=jnp.float32) + bl2[...]
    logits_ref[...] = logits
    pred_ref[...] = pred
    m = jnp.max(logits, axis=-1, keepdims=True)
    lse = jnp.log(jnp.sum(jnp.exp(logits - m), axis=-1, keepdims=True)) + m
    onehot = jax.lax.broadcasted_iota(jnp.int32, (N, NUM_CLASSES), 1) == lab_ref[...]
    picked = jnp.sum(jnp.where(onehot, logits, 0.0), axis=-1, keepdims=True)
    cls = jnp.mean(lse - picked)
    reg = jnp.mean(jnp.abs(pred - bbox_ref[...]))
    cls_ref[...] = cls[None, None]
    reg_ref[...] = reg[None, None]
    tot_ref[...] = (cls + reg)[None, None]


def _tail(x, parts, bbox, labels2d, Wg1, bg1, Wg2, bg2,
          Wc1, bc1, Wc2, bc2, Wl1, bl1, Wl2, bl2):
    return pl.pallas_call(
        _tail_body,
        out_shape=(
            jax.ShapeDtypeStruct((N, NUM_CLASSES), jnp.float32),
            jax.ShapeDtypeStruct((N, 4), jnp.float32),
            jax.ShapeDtypeStruct((1, 1), jnp.float32),
            jax.ShapeDtypeStruct((1, 1), jnp.float32),
            jax.ShapeDtypeStruct((1, 1), jnp.float32),
        ),
    )(x, parts, bbox, labels2d, Wg1, bg1, Wg2, bg2,
      Wc1, bc1, Wc2, bc2, Wl1, bl1, Wl2, bl2)


def kernel(images, layer_rect, edges, bbox, labels, node_indices,
           W_pos, b_pos, W_img, b_img, W_edge, b_edge,
           W_g1, b_g1, W_g2, b_g2,
           W_c1, b_c1, W_c2, b_c2,
           W_l1, b_l1, W_l2, b_l2):
    xi = edges[0, :].astype(jnp.int32)
    xj = edges[1, :].astype(jnp.int32)
    pad = EP - E
    xi_w = jnp.concatenate([xi, jnp.zeros((pad,), jnp.int32)]).reshape(NW, CH, CSZ)
    xj_w = jnp.concatenate([xj, jnp.full((pad,), N, jnp.int32)]).reshape(NW, CH, CSZ)

    rect_flat = layer_rect.reshape(4 * N)

    x = _embed(layer_rect, images, W_pos, W_img, (b_pos + b_img).reshape(1, D))
    dT = _edge_dist(rect_flat, xi_w, xj_w)
    e = _edge_mlp(dT, W_edge, b_edge.reshape(1, D))
    parts = _msg_pass(x, e, xi_w, xj_w)

    logits, pred, tot, cls, reg = _tail(
        x, parts, bbox, labels.astype(jnp.int32).reshape(N, 1),
        W_g1, b_g1.reshape(1, D), W_g2, b_g2.reshape(1, D),
        W_c1, b_c1.reshape(1, D), W_c2, b_c2.reshape(1, NUM_CLASSES),
        W_l1, b_l1.reshape(1, D), W_l2, b_l2.reshape(1, 4))
    return (logits, pred, tot.reshape(()), cls.reshape(()), reg.reshape(()))


# speedup vs baseline: 4.6245x; 1.2167x over previous
import functools

import jax
import jax.numpy as jnp
from jax import lax
from jax.experimental import pallas as pl
from jax.experimental.pallas import tpu as pltpu
from jax.experimental.pallas import tpu_sc as plsc

N = 10000
E = 320000
D = 128
NUM_CLASSES = 25

NC = 2
NS = 16
NW = NC * NS
CSZ = 128
CH = 79
EW = CH * CSZ
EP = NW * EW
AGG_ROWS = 10112
ROWS_PER_TILE = AGG_ROWS // NS

@functools.cache
def _sc_mesh():
    return plsc.VectorSubcoreMesh(core_axis_name="c", subcore_axis_name="s",
                                  num_cores=NC, num_subcores=NS)


def _edge_dist_body(rect_hbm, xi_hbm, xj_hbm, dT_hbm,
                    rect_v, xi_v, xj_v, dbuf):
    wid = lax.axis_index("c") * NS + lax.axis_index("s")
    pltpu.sync_copy(rect_hbm, rect_v)
    pltpu.sync_copy(xi_hbm.at[wid], xi_v)
    pltpu.sync_copy(xj_hbm.at[wid], xj_v)

    def row_body(r, carry):
        for j in range(8):
            s = slice(j * 16, (j + 1) * 16)
            vi = xi_v[r, s] * 4
            vj = xj_v[r, s] * 4
            for c in range(4):
                a = plsc.load_gather(rect_v, [vi + c])
                b = plsc.load_gather(rect_v, [vj + c])
                dbuf[c, pl.ds(r * CSZ + j * 16, 16)] = jnp.abs(a - b)
        return carry

    lax.fori_loop(0, CH, row_body, 0)
    pltpu.sync_copy(dbuf, dT_hbm.at[:, pl.ds(wid * EW, EW)])


def _edge_dist(rect_flat, xi_w, xj_w):
    return pl.kernel(
        _edge_dist_body,
        out_type=jax.ShapeDtypeStruct((4, EP), jnp.float32),
        mesh=_sc_mesh(),
        scratch_types=[
            pltpu.VMEM((4 * N,), jnp.float32),
            pltpu.VMEM((CH, CSZ), jnp.int32),
            pltpu.VMEM((CH, CSZ), jnp.int32),
            pltpu.VMEM((4, EW), jnp.float32),
        ],
        compiler_params=pltpu.CompilerParams(needs_layout_passes=False),
    )(rect_flat, xi_w, xj_w)


def _msg_pass_body(x_hbm, e_hbm, xi_hbm, xj_hbm, parts_hbm,
                   xi2, xj2, xg2, ev, aggr_sh, sem0, sem1):
    core = lax.axis_index("c")
    sub = lax.axis_index("s")
    wid = core * NS + sub
    sems = (sem0, sem1)

    zeros16 = jnp.zeros((16,), jnp.float32)

    def zrow(r, carry):
        for j in range(8):
            ev[r, j * 16:(j + 1) * 16] = zeros16
        return carry

    lax.fori_loop(0, CSZ, zrow, 0)
    base = sub * ROWS_PER_TILE
    for b in range(ROWS_PER_TILE // CSZ):
        pltpu.sync_copy(ev, aggr_sh.at[pl.ds(base + b * CSZ, CSZ)])
    rem = ROWS_PER_TILE % CSZ
    if rem:
        pltpu.sync_copy(ev.at[pl.ds(0, rem)],
                        aggr_sh.at[pl.ds(base + (ROWS_PER_TILE // CSZ) * CSZ, rem)])
    plsc.subcore_barrier()

    def _issue(c, slot, sem):
        pltpu.sync_copy(xi_hbm.at[wid, pl.ds(c, 1)], xi2.at[pl.ds(slot, 1)])
        pltpu.sync_copy(xj_hbm.at[wid, pl.ds(c, 1)], xj2.at[pl.ds(slot, 1)])
        pltpu.async_copy(x_hbm.at[xi2.at[slot]], xg2.at[slot], sem)

    def _consume(c, slot, sem):
        pltpu.make_async_copy(x_hbm.at[xi2.at[slot]], xg2.at[slot], sem).wait()
        pltpu.sync_copy(e_hbm.at[pl.ds(wid * EW + c * CSZ, CSZ)], ev)

        def row_body(r, rcarry):
            for j in range(8):
                s = slice(j * 16, (j + 1) * 16)
                ev[r, s] = jnp.maximum(ev[r, s] + xg2[slot, r, s], 0.0)
            return rcarry

        lax.fori_loop(0, CSZ, row_body, 0)
        pltpu.sync_copy(ev, aggr_sh.at[xj2.at[slot]], add=True)

    _issue(0, 0, sems[0])

    def pair_body(g, carry):
        for b in range(2):
            c = g * 2 + b

            @pl.when(c < CH)
            def _():
                @pl.when(c + 1 < CH)
                def _():
                    _issue(c + 1, (b + 1) % 2, sems[(b + 1) % 2])

                _consume(c, b, sems[b])
        return carry

    lax.fori_loop(0, (CH + 1) // 2, pair_body, 0)
    plsc.subcore_barrier()
    pltpu.sync_copy(aggr_sh.at[pl.ds(base, ROWS_PER_TILE)],
                    parts_hbm.at[core, pl.ds(base, ROWS_PER_TILE)])


def _msg_pass(x, e, xi_w, xj_w):
    return pl.kernel(
        _msg_pass_body,
        out_type=jax.ShapeDtypeStruct((NC, AGG_ROWS, D), jnp.float32),
        mesh=_sc_mesh(),
        scratch_types=[
            pltpu.VMEM((2, CSZ), jnp.int32),
            pltpu.VMEM((2, CSZ), jnp.int32),
            pltpu.VMEM((2, CSZ, D), jnp.float32),
            pltpu.VMEM((CSZ, D), jnp.float32),
            pltpu.VMEM_SHARED((AGG_ROWS, D), jnp.float32),
            pltpu.SemaphoreType.DMA,
            pltpu.SemaphoreType.DMA,
        ],
        compiler_params=pltpu.CompilerParams(needs_layout_passes=False),
    )(x, e, xi_w, xj_w)


def _pos_embed_body(rectT_ref, Wp_ref, x0_ref):
    r = rectT_ref[...]
    feats = [r]
    for i in range(6):
        f = 2.0 ** i
        feats.append(jnp.sin(r * f))
        feats.append(jnp.cos(r * f))
    ft = jnp.concatenate(feats, axis=0)
    x0_ref[...] = lax.dot_general(ft, Wp_ref[...], (((0,), (0,)), ((), ())),
                                  preferred_element_type=jnp.float32)


def _pos_embed(rectT, W_pos):
    return pl.pallas_call(
        _pos_embed_body,
        out_shape=jax.ShapeDtypeStruct((N, D), jnp.float32),
    )(rectT, W_pos)


def _embed_body(x0_ref, img_ref, Wi_ref, b_ref, x_ref):
    img = jnp.dot(img_ref[...], Wi_ref[...], preferred_element_type=jnp.float32)
    x_ref[...] = x0_ref[...] + img + b_ref[...]


def _embed(x0, images, W_img, b):
    BR = 400
    return pl.pallas_call(
        _embed_body,
        grid=(N // BR,),
        in_specs=[
            pl.BlockSpec((BR, D), lambda i: (i, 0)),
            pl.BlockSpec((BR, 768), lambda i: (i, 0)),
            pl.BlockSpec((768, D), lambda i: (0, 0)),
            pl.BlockSpec((1, D), lambda i: (0, 0)),
        ],
        out_specs=pl.BlockSpec((BR, D), lambda i: (i, 0)),
        out_shape=jax.ShapeDtypeStruct((N, D), jnp.float32),
    )(x0, images, W_img, b)


def _edge_mlp_body(dT_ref, W_ref, b_ref, e_ref):
    d = dT_ref[...]
    feats = [d]
    for i in range(3):
        f = 2.0 ** i
        feats.append(jnp.sin(d * f))
        feats.append(jnp.cos(d * f))
    ft = jnp.concatenate(feats, axis=0)
    e_ref[...] = lax.dot_general(ft, W_ref[...], (((0,), (0,)), ((), ())),
                                 preferred_element_type=jnp.float32) + b_ref[...]


def _edge_mlp(dT, W_edge, b_edge):
    BC = 1024
    return pl.pallas_call(
        _edge_mlp_body,
        grid=(EP // BC,),
        in_specs=[
            pl.BlockSpec((4, BC), lambda i: (0, i)),
            pl.BlockSpec((28, D), lambda i: (0, 0)),
            pl.BlockSpec((1, D), lambda i: (0, 0)),
        ],
        out_specs=pl.BlockSpec((BC, D), lambda i: (i, 0)),
        out_shape=jax.ShapeDtypeStruct((EP, D), jnp.float32),
    )(dT, W_edge, b_edge)


def _tail_body(x_ref, parts_ref, bbox_ref, lab_ref,
               Wg1, bg1, Wg2, bg2, Wc1, bc1, Wc2, bc2, Wl1, bl1, Wl2, bl2,
               logits_ref, pred_ref, tot_ref, cls_ref, reg_ref):
    x = x_ref[...]
    aggr = parts_ref[0, :N, :] + parts_ref[1, :N, :]
    h = jnp.maximum(jnp.dot(x + aggr, Wg1[...], preferred_element_type=jnp.float32)
                    + bg1[...], 0.0)
    gnn = jnp.dot(h, Wg2[...], preferred_element_type=jnp.float32) + bg2[...]
    ch = jnp.maximum(jnp.dot(gnn, Wc1[...], preferred_element_type=jnp.float32)
                     + bc1[...], 0.0)
    logits = jnp.dot(ch, Wc2[...], preferred_element_type=jnp.float32) + bc2[...]
    lh = jnp.maximum(jnp.dot(gnn, Wl1[...], preferred_element_type=jnp.float32)
                     + bl1[...], 0.0)
    pred = jnp.dot(lh, Wl2[...], preferred_element_type=jnp.float32) + bl2[...]
    logits_ref[...] = logits
    pred_ref[...] = pred
    m = jnp.max(logits, axis=-1, keepdims=True)
    lse = jnp.log(jnp.sum(jnp.exp(logits - m), axis=-1, keepdims=True)) + m
    onehot = jax.lax.broadcasted_iota(jnp.int32, (N, NUM_CLASSES), 1) == lab_ref[...]
    picked = jnp.sum(jnp.where(onehot, logits, 0.0), axis=-1, keepdims=True)
    cls = jnp.mean(lse - picked)
    reg = jnp.mean(jnp.abs(pred - bbox_ref[...]))
    cls_ref[...] = cls[None, None]
    reg_ref[...] = reg[None, None]
    tot_ref[...] = (cls + reg)[None, None]


def _tail(x, parts, bbox, labels2d, Wg1, bg1, Wg2, bg2,
          Wc1, bc1, Wc2, bc2, Wl1, bl1, Wl2, bl2):
    return pl.pallas_call(
        _tail_body,
        out_shape=(
            jax.ShapeDtypeStruct((N, NUM_CLASSES), jnp.float32),
            jax.ShapeDtypeStruct((N, 4), jnp.float32),
            jax.ShapeDtypeStruct((1, 1), jnp.float32),
            jax.ShapeDtypeStruct((1, 1), jnp.float32),
            jax.ShapeDtypeStruct((1, 1), jnp.float32),
        ),
    )(x, parts, bbox, labels2d, Wg1, bg1, Wg2, bg2,
      Wc1, bc1, Wc2, bc2, Wl1, bl1, Wl2, bl2)


def kernel(images, layer_rect, edges, bbox, labels, node_indices,
           W_pos, b_pos, W_img, b_img, W_edge, b_edge,
           W_g1, b_g1, W_g2, b_g2,
           W_c1, b_c1, W_c2, b_c2,
           W_l1, b_l1, W_l2, b_l2):
    xi = edges[0, :].astype(jnp.int32)
    xj = edges[1, :].astype(jnp.int32)
    pad = EP - E
    xi_w = jnp.concatenate([xi, jnp.zeros((pad,), jnp.int32)]).reshape(NW, CH, CSZ)
    xj_w = jnp.concatenate([xj, jnp.full((pad,), N, jnp.int32)]).reshape(NW, CH, CSZ)

    rect_flat = layer_rect.reshape(4 * N)

    x0 = _pos_embed(layer_rect.T, W_pos)
    x = _embed(x0, images, W_img, (b_pos + b_img).reshape(1, D))
    dT = _edge_dist(rect_flat, xi_w, xj_w)
    e = _edge_mlp(dT, W_edge, b_edge.reshape(1, D))
    parts = _msg_pass(x, e, xi_w, xj_w)

    logits, pred, tot, cls, reg = _tail(
        x, parts, bbox, labels.astype(jnp.int32).reshape(N, 1),
        W_g1, b_g1.reshape(1, D), W_g2, b_g2.reshape(1, D),
        W_c1, b_c1.reshape(1, D), W_c2, b_c2.reshape(1, NUM_CLASSES),
        W_l1, b_l1.reshape(1, D), W_l2, b_l2.reshape(1, 4))
    return (logits, pred, tot.reshape(()), cls.reshape(()), reg.reshape(()))
